# Initial kernel scaffold; baseline (speedup 1.0000x reference)
import jax
import jax.numpy as jnp
from jax.experimental import pallas as pl


def kernel(x, edge_index, W1_src, W1_dst, att1_src, att1_dst, b1, W2_src, W2_dst, att2_src, att2_dst, b2):
    raise NotImplementedError("write your pallas kernel here")


# speedup vs baseline: 21.4201x; 21.4201x over previous
import functools

import jax
import jax.numpy as jnp
from jax import lax
from jax.experimental import pallas as pl
from jax.experimental.pallas import tpu as pltpu
from jax.experimental.pallas import tpu_sc as plsc

N = 10000
D = 128
E = 320000
NCORES = 2
NSUB = 16
NTILES = NCORES * NSUB
EPT = E // NTILES
CHUNKS = EPT // 16
ROWS = (EPT + 127) // 128
EPAD = ROWS * 128
BN = 1000


def _tc_entry_body(x_ref, ws_ref, wd_ref, ats_ref, atd_ref,
                   h_ref, as_ref, ad_ref):
    xb = x_ref[...]
    h = jnp.dot(xb, ws_ref[...], preferred_element_type=jnp.float32)
    h_ref[...] = h
    as_ref[...] = jnp.sum(h * ats_ref[...][None, :], axis=1)[None, None, :]
    hd = jnp.dot(xb, wd_ref[...], preferred_element_type=jnp.float32)
    ad_ref[...] = jnp.sum(hd * atd_ref[...][None, :], axis=1)[None, None, :]


def _tc_entry(x, ws, wd, ats, atd):
    return pl.pallas_call(
        _tc_entry_body,
        grid=(N // BN,),
        in_specs=[
            pl.BlockSpec((BN, D), lambda i: (i, 0)),
            pl.BlockSpec((D, D), lambda i: (0, 0)),
            pl.BlockSpec((D, D), lambda i: (0, 0)),
            pl.BlockSpec((D,), lambda i: (0,)),
            pl.BlockSpec((D,), lambda i: (0,)),
        ],
        out_specs=[
            pl.BlockSpec((BN, D), lambda i: (i, 0)),
            pl.BlockSpec((1, 1, BN), lambda i: (i, 0, 0)),
            pl.BlockSpec((1, 1, BN), lambda i: (i, 0, 0)),
        ],
        out_shape=[
            jax.ShapeDtypeStruct((N, D), jnp.float32),
            jax.ShapeDtypeStruct((N // BN, 1, BN), jnp.float32),
            jax.ShapeDtypeStruct((N // BN, 1, BN), jnp.float32),
        ],
    )(x, ws, wd, ats, atd)


def _tc_mid_body(acc_ref, den_ref, b_ref, ws_ref, wd_ref, ats_ref, atd_ref,
                 h_ref, as_ref, ad_ref):
    den = den_ref[0, 0, 0] + den_ref[1, 0, 0] + 1e-16
    h1 = (acc_ref[0] + acc_ref[1]) / den[:, None] + b_ref[...][None, :]
    h1 = jnp.maximum(h1, 0.0)
    h2 = jnp.dot(h1, ws_ref[...], preferred_element_type=jnp.float32)
    h_ref[...] = h2
    as_ref[...] = jnp.sum(h2 * ats_ref[...][None, :], axis=1)[None, None, :]
    hd = jnp.dot(h1, wd_ref[...], preferred_element_type=jnp.float32)
    ad_ref[...] = jnp.sum(hd * atd_ref[...][None, :], axis=1)[None, None, :]


def _tc_mid(acc, den, b, ws, wd, ats, atd):
    return pl.pallas_call(
        _tc_mid_body,
        grid=(N // BN,),
        in_specs=[
            pl.BlockSpec((2, BN, D), lambda i: (0, i, 0)),
            pl.BlockSpec((2, 1, 1, BN), lambda i: (0, i, 0, 0)),
            pl.BlockSpec((D,), lambda i: (0,)),
            pl.BlockSpec((D, D), lambda i: (0, 0)),
            pl.BlockSpec((D, D), lambda i: (0, 0)),
            pl.BlockSpec((D,), lambda i: (0,)),
            pl.BlockSpec((D,), lambda i: (0,)),
        ],
        out_specs=[
            pl.BlockSpec((BN, D), lambda i: (i, 0)),
            pl.BlockSpec((1, 1, BN), lambda i: (i, 0, 0)),
            pl.BlockSpec((1, 1, BN), lambda i: (i, 0, 0)),
        ],
        out_shape=[
            jax.ShapeDtypeStruct((N, D), jnp.float32),
            jax.ShapeDtypeStruct((N // BN, 1, BN), jnp.float32),
            jax.ShapeDtypeStruct((N // BN, 1, BN), jnp.float32),
        ],
    )(acc, den, b, ws, wd, ats, atd)


def _tc_out_body(acc_ref, den_ref, b_ref, o_ref):
    den = den_ref[0, 0, 0] + den_ref[1, 0, 0] + 1e-16
    o_ref[...] = (acc_ref[0] + acc_ref[1]) / den[:, None] + b_ref[...][None, :]


def _tc_out(acc, den, b):
    return pl.pallas_call(
        _tc_out_body,
        grid=(N // BN,),
        in_specs=[
            pl.BlockSpec((2, BN, D), lambda i: (0, i, 0)),
            pl.BlockSpec((2, 1, 1, BN), lambda i: (0, i, 0, 0)),
            pl.BlockSpec((D,), lambda i: (0,)),
        ],
        out_specs=pl.BlockSpec((BN, D), lambda i: (i, 0)),
        out_shape=jax.ShapeDtypeStruct((N, D), jnp.float32),
    )(acc, den, b)


def _sc_edge_body(h_hbm, as_hbm, ad_hbm, srcp_hbm, dstp_hbm, zr_hbm, z1_hbm,
                  accp_hbm, denp_hbm,
                  srcv, dstv, exv, rows, av, bv, out_sp, den_sp, sem):
    core = lax.axis_index("c")
    sub = lax.axis_index("s")
    wid = core * NSUB + sub

    @pl.when(sub < 10)
    def _():
        pltpu.sync_copy(zr_hbm.at[pl.ds(sub * 1000, 1000)],
                        out_sp.at[pl.ds(sub * 1000, 1000)])

    @pl.when(sub == 0)
    def _():
        pltpu.sync_copy(z1_hbm, den_sp)

    pltpu.sync_copy(srcp_hbm.at[wid], srcv)
    pltpu.sync_copy(dstp_hbm.at[wid], dstv)
    plsc.subcore_barrier()

    lanes = lax.iota(jnp.int32, 16)

    def p2(j, carry):
        cp = pltpu.async_copy(h_hbm.at[srcv.at[j]], rows, sem)
        pltpu.sync_copy(as_hbm.at[srcv.at[j]], av)
        pltpu.sync_copy(ad_hbm.at[dstv.at[j]], bv)
        nvalid = EPT - j * 128
        for g in range(8):
            sl = pl.ds(g * 16, 16)
            al = av[sl] + bv[sl]
            al = jnp.where(al >= 0.0, al, 0.2 * al)
            e = jnp.exp(al)
            e = jnp.where(lanes + (g * 16) < nvalid, e, 0.0)
            exv[j, sl] = e
        pltpu.sync_copy(exv.at[j], den_sp.at[dstv.at[j]], add=True)
        cp.wait()

        def scale_row(rr, c2):
            e = plsc.load_gather(exv, [jnp.full((16,), j, jnp.int32),
                                       jnp.full((16,), rr, jnp.int32)])
            for c8 in range(8):
                sl = pl.ds(c8 * 16, 16)
                rows[rr, sl] = rows[rr, sl] * e
            return c2

        lax.fori_loop(0, 128, scale_row, 0)
        pltpu.sync_copy(rows, out_sp.at[dstv.at[j]], add=True)
        return carry

    lax.fori_loop(0, ROWS, p2, 0)

    plsc.subcore_barrier()

    @pl.when(sub < 10)
    def _():
        pltpu.sync_copy(out_sp.at[pl.ds(sub * 1000, 1000)],
                        accp_hbm.at[core, pl.ds(sub * 1000, 1000)])

    @pl.when(sub == 0)
    def _():
        pltpu.sync_copy(den_sp, denp_hbm.at[core])


_sc_edge = pl.kernel(
    _sc_edge_body,
    out_type=[
        jax.ShapeDtypeStruct((NCORES, N, D), jnp.float32),
        jax.ShapeDtypeStruct((NCORES, N), jnp.float32),
    ],
    mesh=plsc.VectorSubcoreMesh(core_axis_name="c", subcore_axis_name="s",
                                num_cores=NCORES, num_subcores=NSUB),
    scratch_types=[
        pltpu.VMEM((ROWS, 128), jnp.int32),
        pltpu.VMEM((ROWS, 128), jnp.int32),
        pltpu.VMEM((ROWS, 128), jnp.float32),
        pltpu.VMEM((128, D), jnp.float32),
        pltpu.VMEM((128,), jnp.float32),
        pltpu.VMEM((128,), jnp.float32),
        pltpu.VMEM_SHARED((N, D), jnp.float32),
        pltpu.VMEM_SHARED((N,), jnp.float32),
        pltpu.SemaphoreType.DMA,
    ],
    compiler_params=pltpu.CompilerParams(needs_layout_passes=False),
)


def kernel(x, edge_index, W1_src, W1_dst, att1_src, att1_dst, b1,
           W2_src, W2_dst, att2_src, att2_dst, b2):
    src = edge_index[0].astype(jnp.int32).reshape(NTILES, EPT)
    dst = edge_index[1].astype(jnp.int32).reshape(NTILES, EPT)
    srcp = jnp.pad(src, ((0, 0), (0, EPAD - EPT))).reshape(NTILES, ROWS, 128)
    dstp = jnp.pad(dst, ((0, 0), (0, EPAD - EPT))).reshape(NTILES, ROWS, 128)
    zr = jnp.zeros((N, D), jnp.float32)
    z1 = jnp.zeros((N,), jnp.float32)

    h1, a1s, a1d = _tc_entry(x, W1_src, W1_dst, att1_src, att1_dst)
    acc1, den1 = _sc_edge(h1, a1s.reshape(N), a1d.reshape(N), srcp, dstp, zr, z1)
    h2, a2s, a2d = _tc_mid(acc1, den1.reshape(2, N // BN, 1, BN), b1,
                           W2_src, W2_dst, att2_src, att2_dst)
    acc2, den2 = _sc_edge(h2, a2s.reshape(N), a2d.reshape(N), srcp, dstp, zr, z1)
    return _tc_out(acc2, den2.reshape(2, N // BN, 1, BN), b2)
